# Initial kernel scaffold; baseline (speedup 1.0000x reference)
import functools

import jax
import jax.numpy as jnp
from jax import lax
from jax.experimental import pallas as pl
from jax.experimental.pallas import tpu as pltpu
from jax.experimental.pallas import tpu_sc as plsc

L = 16


def kernel(lS_i, lS_o, W):
    del lS_o
    n_fields, batch = lS_i.shape
    _, vocab, dim = W.shape

    info = plsc.get_sparse_core_info()
    nw = info.num_cores * info.num_subcores
    b_per_w = batch // nw
    rows_per_w = b_per_w * n_fields
    n_chunks = rows_per_w // b_per_w

    mesh = plsc.VectorSubcoreMesh(core_axis_name="c", subcore_axis_name="s")
    w_flat = W.reshape(n_fields * vocab, dim)

    @functools.partial(
        pl.kernel,
        out_type=jax.ShapeDtypeStruct((batch * n_fields, dim), jnp.float32),
        mesh=mesh,
        scratch_types=[
            pltpu.VMEM((n_fields, b_per_w), jnp.int32),
            pltpu.VMEM((n_chunks, b_per_w), jnp.int32),
            pltpu.VMEM((rows_per_w, dim), jnp.float32),
            pltpu.SemaphoreType.DMA,
        ],
    )
    def sc_kernel(idx_hbm, table_hbm, out_hbm, idx_stage, gather_idx, rows, sem):
        wid = lax.axis_index("s") * info.num_cores + lax.axis_index("c")
        b0 = wid * b_per_w

        pltpu.sync_copy(idx_hbm.at[:, pl.ds(b0, b_per_w)], idx_stage)

        lane = lax.iota(jnp.int32, L)

        def field_body(f, _):
            def vec_body(i, _):
                v = idx_stage[f, pl.ds(i * L, L)] + f * vocab
                pos = (i * L + lane) * n_fields + f
                plsc.store_scatter(
                    gather_idx,
                    [lax.shift_right_logical(pos, 7), lax.bitwise_and(pos, 127)],
                    v,
                )
                return 0

            lax.fori_loop(0, b_per_w // L, vec_body, 0)
            return 0

        lax.fori_loop(0, n_fields, field_body, 0)

        def fire(j, _):
            pltpu.async_copy(
                table_hbm.at[gather_idx.at[j]],
                rows.at[pl.ds(j * b_per_w, b_per_w)],
                sem,
            )
            return 0

        lax.fori_loop(0, n_chunks, fire, 0)
        pltpu.make_async_copy(
            out_hbm.at[pl.ds(b0 * n_fields, rows_per_w)], rows, sem
        ).wait()

        pltpu.sync_copy(rows, out_hbm.at[pl.ds(b0 * n_fields, rows_per_w)])

    out = sc_kernel(lS_i, w_flat)
    return out.reshape(batch, n_fields, dim)


# speedup vs baseline: 1.0394x; 1.0394x over previous
import functools

import jax
import jax.numpy as jnp
from jax import lax
from jax.experimental import pallas as pl
from jax.experimental.pallas import tpu as pltpu
from jax.experimental.pallas import tpu_sc as plsc

CHUNK = 10240


def kernel(lS_i, lS_o, W):
    del lS_o
    n_fields, batch = lS_i.shape
    _, vocab, dim = W.shape

    info = plsc.get_sparse_core_info()
    nw = info.num_cores * info.num_subcores
    d_oct = dim // 8
    nblocks = n_fields * d_oct
    tasks_per_w = (nblocks + nw - 1) // nw
    nvec = batch // 16

    aligned_end = (vocab // 128) * 128
    sizes = [CHUNK] * (aligned_end // CHUNK)
    if aligned_end % CHUNK:
        sizes.append(aligned_end % CHUNK)
    tail_lo = vocab - 128

    wt = jnp.transpose(W, (0, 2, 1))
    idx_flat = lS_i.reshape(n_fields * batch)
    tail_t = jnp.transpose(W[:, tail_lo:, :], (0, 2, 1))

    mesh = plsc.VectorSubcoreMesh(core_axis_name="c", subcore_axis_name="s")

    @functools.partial(
        pl.kernel,
        out_type=jax.ShapeDtypeStruct((nblocks * 8, batch), jnp.float32),
        mesh=mesh,
        compiler_params=pltpu.CompilerParams(needs_layout_passes=False),
        scratch_types=[
            pltpu.VMEM((8, CHUNK), jnp.float32),
            pltpu.VMEM((batch,), jnp.int32),
            pltpu.VMEM((8, batch), jnp.float32),
        ],
    )
    def sc_kernel(idx_hbm, table_hbm, tail_hbm, out_hbm, buf, idx_v, acc):
        wid = lax.axis_index("s") * info.num_cores + lax.axis_index("c")
        zero16 = jnp.zeros((16,), jnp.float32)

        def task(t, _):
            block = wid + t * nw

            @pl.when(block < nblocks)
            def _():
                f = lax.shift_right_logical(block, 2)
                db = lax.bitwise_and(block, 3)
                pltpu.sync_copy(idx_hbm.at[pl.ds(f * batch, batch)], idx_v)

                def zero(i, _):
                    for d in range(8):
                        acc[d, pl.ds(i * 16, 16)] = zero16
                    return 0

                lax.fori_loop(0, nvec, zero, 0)

                def sweep_chunk(lo, hi, base, src_buf, sz):
                    def sweep(i, _):
                        iv = idx_v[pl.ds(i * 16, 16)]
                        inb = jnp.logical_and(iv >= lo, iv < hi)
                        loc = jnp.clip(iv - base, 0, sz - 1)
                        nhit = plsc.all_reduce_population_count(inb)[0]

                        @pl.when(nhit > 0)
                        def _():
                            for d in range(8):
                                dv = jnp.full((16,), d, jnp.int32)
                                v = plsc.load_gather(src_buf, [dv, loc])
                                plsc.addupdate(
                                    acc.at[d, pl.ds(i * 16, 16)],
                                    jnp.where(inb, v, 0.0),
                                )

                        return 0

                    lax.fori_loop(0, nvec, sweep, 0)

                c0 = 0
                for sz in sizes:
                    pltpu.sync_copy(
                        table_hbm.at[f, pl.ds(db * 8, 8), pl.ds(c0, sz)],
                        buf.at[:, pl.ds(0, sz)],
                    )
                    sweep_chunk(c0, c0 + sz, c0, buf.at[:, pl.ds(0, sz)], sz)
                    c0 += sz

                if vocab != aligned_end:
                    pltpu.sync_copy(
                        tail_hbm.at[f, pl.ds(db * 8, 8), :],
                        buf.at[:, pl.ds(0, 128)],
                    )
                    sweep_chunk(
                        aligned_end, vocab, tail_lo,
                        buf.at[:, pl.ds(0, 128)], 128,
                    )

                pltpu.sync_copy(acc, out_hbm.at[pl.ds(block * 8, 8), :])

            return 0

        lax.fori_loop(0, tasks_per_w, task, 0)

    out_t = sc_kernel(idx_flat, wt, tail_t)
    return jnp.transpose(out_t.reshape(n_fields, dim, batch), (2, 0, 1))


# speedup vs baseline: 2.4989x; 2.4040x over previous
import functools

import jax
import jax.numpy as jnp
from jax import lax
from jax.experimental import pallas as pl
from jax.experimental.pallas import tpu as pltpu
from jax.experimental.pallas import tpu_sc as plsc

CHUNK = 5376


def kernel(lS_i, lS_o, W):
    del lS_o
    n_fields, batch = lS_i.shape
    _, vocab, dim = W.shape

    info = plsc.get_sparse_core_info()
    nw = info.num_cores * info.num_subcores
    d_oct = dim // 8
    nblocks = n_fields * d_oct
    tasks_per_w = (nblocks + nw - 1) // nw

    aligned_end = (vocab // 128) * 128
    sizes = [CHUNK] * (aligned_end // CHUNK)
    if aligned_end % CHUNK:
        sizes.append(aligned_end % CHUNK)
    tail_lo = vocab - 128
    n_sweeps = len(sizes) + 1

    bag_iota = lax.broadcasted_iota(jnp.int32, (n_fields, batch), 1)
    s_idx, s_bag = lax.sort((lS_i, bag_iota), dimension=1, num_keys=1)
    bounds = []
    c0 = 0
    for sz in sizes:
        bounds.append(c0)
        c0 += sz
    bounds.append(aligned_end)
    bounds_a = jnp.array(bounds, dtype=jnp.int32)
    starts = jax.vmap(
        lambda row: jnp.searchsorted(row, bounds_a).astype(jnp.int32)
    )(s_idx)
    n_st = ((n_sweeps + 1 + 15) // 16) * 16
    starts = jnp.concatenate(
        [
            starts,
            jnp.full((n_fields, n_st - n_sweeps), batch, jnp.int32),
        ],
        axis=1,
    )

    wt = jnp.transpose(W, (0, 2, 1))
    tail_t = jnp.transpose(W[:, tail_lo:, :], (0, 2, 1))
    s_idx_f = s_idx.reshape(n_fields * batch)
    s_bag_f = s_bag.reshape(n_fields * batch)
    starts_f = starts.reshape(n_fields * n_st)

    mesh = plsc.VectorSubcoreMesh(core_axis_name="c", subcore_axis_name="s")

    @functools.partial(
        pl.kernel,
        out_type=jax.ShapeDtypeStruct((nblocks * 8, batch), jnp.float32),
        mesh=mesh,
        compiler_params=pltpu.CompilerParams(needs_layout_passes=False),
        scratch_types=[
            pltpu.VMEM((2, 8, CHUNK), jnp.float32),
            pltpu.VMEM((batch + 16,), jnp.int32),
            pltpu.VMEM((batch + 16,), jnp.int32),
            pltpu.VMEM((n_st,), jnp.int32),
            pltpu.VMEM((8, batch), jnp.float32),
            pltpu.SemaphoreType.DMA,
        ],
    )
    def sc_kernel(sidx_hbm, sbag_hbm, st_hbm, table_hbm, tail_hbm, out_hbm,
                  buf2, sidx_v, sbag_v, st_v, acc, sem):
        wid = lax.axis_index("s") * info.num_cores + lax.axis_index("c")
        lane = lax.iota(jnp.int32, 16)

        def task(t, _):
            block = wid + t * nw

            @pl.when(block < nblocks)
            def _():
                f = lax.shift_right_logical(block, 2)
                db = lax.bitwise_and(block, 3)
                pltpu.sync_copy(
                    sidx_hbm.at[pl.ds(f * batch, batch)],
                    sidx_v.at[pl.ds(0, batch)],
                )
                pltpu.sync_copy(
                    sbag_hbm.at[pl.ds(f * batch, batch)],
                    sbag_v.at[pl.ds(0, batch)],
                )
                pltpu.sync_copy(st_hbm.at[pl.ds(f * n_st, n_st)], st_v)

                def stage(c, slot):
                    if c < len(sizes):
                        return pltpu.async_copy(
                            table_hbm.at[
                                f, pl.ds(db * 8, 8),
                                pl.ds(bounds[c], sizes[c]),
                            ],
                            buf2.at[slot, :, pl.ds(0, sizes[c])],
                            sem,
                        )
                    return pltpu.async_copy(
                        tail_hbm.at[f, pl.ds(db * 8, 8), :],
                        buf2.at[slot, :, pl.ds(0, 128)],
                        sem,
                    )

                def sweep(c, slot, lo, hi):
                    base = bounds[c] if c < len(sizes) else tail_lo
                    sz = sizes[c] if c < len(sizes) else 128
                    src = buf2.at[slot, :, pl.ds(0, sz)]

                    def body(j, _):
                        p0 = lo + j * 16
                        iv = sidx_v[pl.ds(p0, 16)]
                        bagv = sbag_v[pl.ds(p0, 16)]
                        msk = (p0 + lane) < hi
                        loc = jnp.clip(iv - base, 0, sz - 1)
                        for d in range(8):
                            dv = jnp.full((16,), d, jnp.int32)
                            v = plsc.load_gather(src, [dv, loc])
                            plsc.store_scatter(acc, [dv, bagv], v, mask=msk)
                        return 0

                    nvec = lax.shift_right_logical(hi - lo + 15, 4)
                    lax.fori_loop(0, nvec, body, 0)

                handles = [stage(0, 0)]
                sv = [st_v[pl.ds(k * 16, 16)] for k in range(n_st // 16)]

                def bound_at(c):
                    return sv[c // 16][c % 16]

                for c in range(n_sweeps):
                    handles[c].wait()
                    if c + 1 < n_sweeps:
                        handles.append(stage(c + 1, (c + 1) & 1))
                    sweep(c, c & 1, bound_at(c), bound_at(c + 1))

                pltpu.sync_copy(acc, out_hbm.at[pl.ds(block * 8, 8), :])

            return 0

        lax.fori_loop(0, tasks_per_w, task, 0)

    out_t = sc_kernel(s_idx_f, s_bag_f, starts_f, wt, tail_t)
    return jnp.transpose(out_t.reshape(n_fields, dim, batch), (2, 0, 1))


# speedup vs baseline: 2.5605x; 1.0247x over previous
import functools

import jax
import jax.numpy as jnp
from jax import lax
from jax.experimental import pallas as pl
from jax.experimental.pallas import tpu as pltpu
from jax.experimental.pallas import tpu_sc as plsc

CHUNK = 5376


def kernel(lS_i, lS_o, W):
    del lS_o
    n_fields, batch = lS_i.shape
    _, vocab, dim = W.shape

    info = plsc.get_sparse_core_info()
    nw = info.num_cores * info.num_subcores
    d_oct = dim // 8
    nblocks = n_fields * d_oct
    tasks_per_w = (nblocks + nw - 1) // nw

    aligned_end = (vocab // 128) * 128
    sizes = [CHUNK] * (aligned_end // CHUNK)
    if aligned_end % CHUNK:
        sizes.append(aligned_end % CHUNK)
    tail_lo = vocab - 128
    n_sweeps = len(sizes) + 1

    bag_iota = lax.broadcasted_iota(jnp.int32, (n_fields, batch), 1)
    comb = lS_i * batch + bag_iota
    s_comb = lax.sort(comb, dimension=1)
    bounds = []
    c0 = 0
    for sz in sizes:
        bounds.append(c0)
        c0 += sz
    bounds.append(aligned_end)
    bounds_a = jnp.array(bounds, dtype=jnp.int32) * batch
    starts = jax.vmap(
        lambda row: jnp.searchsorted(row, bounds_a).astype(jnp.int32)
    )(s_comb)
    n_st = ((n_sweeps + 1 + 15) // 16) * 16
    starts = jnp.concatenate(
        [
            starts,
            jnp.full((n_fields, n_st - n_sweeps), batch, jnp.int32),
        ],
        axis=1,
    )

    wt = jnp.transpose(W, (0, 2, 1))
    tail_t = jnp.transpose(W[:, tail_lo:, :], (0, 2, 1))
    s_comb_f = s_comb.reshape(n_fields * batch)
    starts_f = starts.reshape(n_fields * n_st)
    bag_shift = batch.bit_length() - 1

    mesh = plsc.VectorSubcoreMesh(core_axis_name="c", subcore_axis_name="s")

    @functools.partial(
        pl.kernel,
        out_type=jax.ShapeDtypeStruct((nblocks * 8, batch), jnp.float32),
        mesh=mesh,
        compiler_params=pltpu.CompilerParams(needs_layout_passes=False),
        scratch_types=[
            pltpu.VMEM((2, 8, CHUNK), jnp.float32),
            pltpu.VMEM((batch + 16,), jnp.int32),
            pltpu.VMEM((n_st,), jnp.int32),
            pltpu.VMEM((8, batch), jnp.float32),
            pltpu.SemaphoreType.DMA,
        ],
    )
    def sc_kernel(scomb_hbm, st_hbm, table_hbm, tail_hbm, out_hbm,
                  buf2, scomb_v, st_v, acc, sem):
        wid = lax.axis_index("s") * info.num_cores + lax.axis_index("c")
        lane = lax.iota(jnp.int32, 16)

        def task(t, _):
            block = wid + t * nw

            @pl.when(block < nblocks)
            def _():
                f = lax.shift_right_logical(block, 2)
                db = lax.bitwise_and(block, 3)
                pltpu.sync_copy(
                    scomb_hbm.at[pl.ds(f * batch, batch)],
                    scomb_v.at[pl.ds(0, batch)],
                )
                pltpu.sync_copy(st_hbm.at[pl.ds(f * n_st, n_st)], st_v)

                def stage(c, slot):
                    if c < len(sizes):
                        return pltpu.async_copy(
                            table_hbm.at[
                                f, pl.ds(db * 8, 8),
                                pl.ds(bounds[c], sizes[c]),
                            ],
                            buf2.at[slot, :, pl.ds(0, sizes[c])],
                            sem,
                        )
                    return pltpu.async_copy(
                        tail_hbm.at[f, pl.ds(db * 8, 8), :],
                        buf2.at[slot, :, pl.ds(0, 128)],
                        sem,
                    )

                def sweep(c, slot, lo, hi):
                    base = bounds[c] if c < len(sizes) else tail_lo
                    sz = sizes[c] if c < len(sizes) else 128
                    src = buf2.at[slot, :, pl.ds(0, sz)]

                    def body(j, _):
                        p0 = lo + j * 16
                        cv = scomb_v[pl.ds(p0, 16)]
                        iv = lax.shift_right_logical(cv, bag_shift)
                        bagv = lax.bitwise_and(cv, batch - 1)
                        msk = (p0 + lane) < hi
                        loc = jnp.clip(iv - base, 0, sz - 1)
                        for d in range(8):
                            dv = jnp.full((16,), d, jnp.int32)
                            v = plsc.load_gather(src, [dv, loc])
                            plsc.store_scatter(acc, [dv, bagv], v, mask=msk)
                        return 0

                    nvec = lax.shift_right_logical(hi - lo + 15, 4)
                    lax.fori_loop(0, nvec, body, 0)

                handles = [stage(0, 0)]
                sv = [st_v[pl.ds(k * 16, 16)] for k in range(n_st // 16)]

                def bound_at(c):
                    return sv[c // 16][c % 16]

                for c in range(n_sweeps):
                    handles[c].wait()
                    if c + 1 < n_sweeps:
                        handles.append(stage(c + 1, (c + 1) & 1))
                    sweep(c, c & 1, bound_at(c), bound_at(c + 1))

                pltpu.sync_copy(acc, out_hbm.at[pl.ds(block * 8, 8), :])

            return 0

        lax.fori_loop(0, tasks_per_w, task, 0)

    out_t = sc_kernel(s_comb_f, starts_f, wt, tail_t)
    return jnp.transpose(out_t.reshape(n_fields, dim, batch), (2, 0, 1))


# speedup vs baseline: 3.2326x; 1.2625x over previous
import functools

import jax
import jax.numpy as jnp
from jax import lax
from jax.experimental import pallas as pl
from jax.experimental.pallas import tpu as pltpu
from jax.experimental.pallas import tpu_sc as plsc

CHUNK = 4096


def kernel(lS_i, lS_o, W):
    del lS_o
    n_fields, batch = lS_i.shape
    _, vocab, dim = W.shape

    info = plsc.get_sparse_core_info()
    nw = info.num_cores * info.num_subcores
    d_oct = dim // 8
    nblocks = n_fields * d_oct
    tasks_per_w = (nblocks + nw - 1) // nw

    aligned_end = (vocab // 128) * 128
    sizes = [CHUNK] * (aligned_end // CHUNK)
    if aligned_end % CHUNK:
        sizes.append(aligned_end % CHUNK)
    tail_lo = vocab - 128
    n_sweeps = len(sizes) + 1

    bag_iota = lax.broadcasted_iota(jnp.int32, (n_fields, batch), 1)
    comb = lS_i * batch + bag_iota
    s_comb = lax.sort(comb, dimension=1)
    bounds = []
    c0 = 0
    for sz in sizes:
        bounds.append(c0)
        c0 += sz
    bounds.append(aligned_end)

    wt = jnp.transpose(W, (0, 2, 1))
    tail_t = jnp.transpose(W[:, tail_lo:, :], (0, 2, 1))
    s_comb_f = s_comb.reshape(n_fields * batch)
    bag_shift = batch.bit_length() - 1

    mesh = plsc.VectorSubcoreMesh(core_axis_name="c", subcore_axis_name="s")

    @functools.partial(
        pl.kernel,
        out_type=jax.ShapeDtypeStruct((nblocks * 8, batch), jnp.float32),
        mesh=mesh,
        compiler_params=pltpu.CompilerParams(needs_layout_passes=False),
        scratch_types=[
            pltpu.VMEM((2, 8, CHUNK), jnp.float32),
            pltpu.VMEM((batch + 16,), jnp.int32),
            pltpu.VMEM((8, batch), jnp.float32),
            pltpu.SemaphoreType.DMA,
        ],
    )
    def sc_kernel(scomb_hbm, table_hbm, tail_hbm, out_hbm,
                  buf2, scomb_v, acc, sem):
        wid = lax.axis_index("s") * info.num_cores + lax.axis_index("c")
        lane = lax.iota(jnp.int32, 16)

        def task(t, _):
            block = wid + t * nw

            @pl.when(block < nblocks)
            def _():
                f = lax.shift_right_logical(block, 2)
                db = lax.bitwise_and(block, 3)
                pltpu.sync_copy(
                    scomb_hbm.at[pl.ds(f * batch, batch)],
                    scomb_v.at[pl.ds(0, batch)],
                )

                def stage(c, slot):
                    if c < len(sizes):
                        return pltpu.async_copy(
                            table_hbm.at[
                                f, pl.ds(db * 8, 8),
                                pl.ds(bounds[c], sizes[c]),
                            ],
                            buf2.at[slot, :, pl.ds(0, sizes[c])],
                            sem,
                        )
                    return pltpu.async_copy(
                        tail_hbm.at[f, pl.ds(db * 8, 8), :],
                        buf2.at[slot, :, pl.ds(0, 128)],
                        sem,
                    )

                def sweep(c, slot, p0):
                    base = bounds[c] if c < len(sizes) else tail_lo
                    sz = sizes[c] if c < len(sizes) else 128
                    lo_cut = bounds[c] * batch
                    hi_cut = (
                        (bounds[c] + sizes[c]) * batch
                        if c < len(sizes) else jnp.int32(2**31 - 1)
                    )
                    src = buf2.at[slot, :, pl.ds(0, sz)]

                    def cond(carry):
                        return carry[1]

                    def body(carry):
                        p, _ = carry
                        cv = scomb_v[pl.ds(p, 16)]
                        below = cv < hi_cut
                        msk = jnp.logical_and(cv >= lo_cut, below)
                        iv = lax.shift_right_logical(cv, bag_shift)
                        bagv = lax.bitwise_and(cv, batch - 1)
                        loc = jnp.clip(iv - base, 0, sz - 1)
                        for d in range(8):
                            dv = jnp.full((16,), d, jnp.int32)
                            v = plsc.load_gather(src, [dv, loc])
                            plsc.store_scatter(acc, [dv, bagv], v, mask=msk)
                        all_in = plsc.all_reduce_population_count(below)[0]
                        p_new = jnp.where(all_in == 16, p + 16, p)
                        cont = jnp.logical_and(all_in == 16, p_new < batch)
                        return (p_new, cont)

                    p_end, _ = lax.while_loop(cond, body, (p0, p0 < batch))
                    return p_end

                handles = [stage(0, 0)]
                pos = jnp.int32(0)
                for c in range(n_sweeps):
                    handles[c].wait()
                    if c + 1 < n_sweeps:
                        handles.append(stage(c + 1, (c + 1) & 1))
                    pos = sweep(c, c & 1, pos)

                pltpu.sync_copy(acc, out_hbm.at[pl.ds(block * 8, 8), :])

            return 0

        lax.fori_loop(0, tasks_per_w, task, 0)

    out_t = sc_kernel(s_comb_f, wt, tail_t)
    return jnp.transpose(out_t.reshape(n_fields, dim, batch), (2, 0, 1))


# speedup vs baseline: 3.3043x; 1.0222x over previous
import functools

import jax
import jax.numpy as jnp
from jax import lax
from jax.experimental import pallas as pl
from jax.experimental.pallas import tpu as pltpu
from jax.experimental.pallas import tpu_sc as plsc

CHUNK = 4096


def kernel(lS_i, lS_o, W):
    del lS_o
    n_fields, batch = lS_i.shape
    _, vocab, dim = W.shape

    info = plsc.get_sparse_core_info()
    nw = info.num_cores * info.num_subcores
    d_oct = dim // 8
    nblocks = n_fields * d_oct
    tasks_per_w = (nblocks + nw - 1) // nw

    aligned_end = (vocab // 128) * 128
    sizes = [CHUNK] * (aligned_end // CHUNK)
    if aligned_end % CHUNK:
        sizes.append(aligned_end % CHUNK)
    tail_lo = vocab - 128
    n_sweeps = len(sizes) + 1

    bag_iota = lax.broadcasted_iota(jnp.int32, (n_fields, batch), 1)
    comb = lS_i * batch + bag_iota
    bounds = []
    c0 = 0
    for sz in sizes:
        bounds.append(c0)
        c0 += sz
    bounds.append(aligned_end)

    wt = jnp.transpose(W, (0, 2, 1))
    tail_t = jnp.transpose(W[:, tail_lo:, :], (0, 2, 1))
    bag_shift = batch.bit_length() - 1

    mesh = plsc.VectorSubcoreMesh(core_axis_name="c", subcore_axis_name="s")

    def make_call(f0, nf):
        nblk = nf * d_oct
        tpw = (nblk + nw - 1) // nw

        @functools.partial(
            pl.kernel,
            out_type=jax.ShapeDtypeStruct((nblk * 8, batch), jnp.float32),
            mesh=mesh,
            compiler_params=pltpu.CompilerParams(needs_layout_passes=False),
            scratch_types=[
                pltpu.VMEM((2, 8, CHUNK), jnp.float32),
                pltpu.VMEM((batch + 16,), jnp.int32),
                pltpu.VMEM((8, batch), jnp.float32),
                pltpu.SemaphoreType.DMA,
            ],
        )
        def sc_kernel(scomb_hbm, table_hbm, tail_hbm, out_hbm,
                      buf2, scomb_v, acc, sem):
            wid = lax.axis_index("s") * info.num_cores + lax.axis_index("c")

            def task(t, _):
                block = wid + t * nw

                @pl.when(block < nblk)
                def _():
                    f = f0 + lax.shift_right_logical(block, 2)
                    db = lax.bitwise_and(block, 3)
                    f_local = lax.shift_right_logical(block, 2)
                    pltpu.sync_copy(
                        scomb_hbm.at[pl.ds(f_local * batch, batch)],
                        scomb_v.at[pl.ds(0, batch)],
                    )

                    def stage(c, slot):
                        if c < len(sizes):
                            return pltpu.async_copy(
                                table_hbm.at[
                                    f, pl.ds(db * 8, 8),
                                    pl.ds(bounds[c], sizes[c]),
                                ],
                                buf2.at[slot, :, pl.ds(0, sizes[c])],
                                sem,
                            )
                        return pltpu.async_copy(
                            tail_hbm.at[f, pl.ds(db * 8, 8), :],
                            buf2.at[slot, :, pl.ds(0, 128)],
                            sem,
                        )

                    def sweep(c, slot, p0):
                        base = bounds[c] if c < len(sizes) else tail_lo
                        sz = sizes[c] if c < len(sizes) else 128
                        lo_cut = bounds[c] * batch
                        hi_cut = (
                            (bounds[c] + sizes[c]) * batch
                            if c < len(sizes) else jnp.int32(2**31 - 1)
                        )
                        src = buf2.at[slot, :, pl.ds(0, sz)]

                        def cond(carry):
                            return carry[1]

                        def body(carry):
                            p, _ = carry
                            cv = scomb_v[pl.ds(p, 16)]
                            below = cv < hi_cut
                            msk = jnp.logical_and(cv >= lo_cut, below)
                            iv = lax.shift_right_logical(cv, bag_shift)
                            bagv = lax.bitwise_and(cv, batch - 1)
                            loc = jnp.clip(iv - base, 0, sz - 1)
                            for d in range(8):
                                dv = jnp.full((16,), d, jnp.int32)
                                v = plsc.load_gather(src, [dv, loc])
                                plsc.store_scatter(
                                    acc, [dv, bagv], v, mask=msk
                                )
                            all_in = plsc.all_reduce_population_count(
                                below
                            )[0]
                            p_new = jnp.where(all_in == 16, p + 16, p)
                            cont = jnp.logical_and(
                                all_in == 16, p_new < batch
                            )
                            return (p_new, cont)

                        p_end, _ = lax.while_loop(
                            cond, body, (p0, p0 < batch)
                        )
                        return p_end

                    handles = [stage(0, 0)]
                    pos = jnp.int32(0)
                    for c in range(n_sweeps):
                        handles[c].wait()
                        if c + 1 < n_sweeps:
                            handles.append(stage(c + 1, (c + 1) & 1))
                        pos = sweep(c, c & 1, pos)

                    pltpu.sync_copy(acc, out_hbm.at[pl.ds(block * 8, 8), :])

                return 0

            lax.fori_loop(0, tpw, task, 0)

        return sc_kernel

    half = n_fields // 2
    s1 = lax.sort(comb[:half], dimension=1).reshape(half * batch)
    s2 = lax.sort(comb[half:], dimension=1).reshape((n_fields - half) * batch)
    out1 = make_call(0, half)(s1, wt, tail_t)
    out2 = make_call(half, n_fields - half)(s2, wt, tail_t)
    out_t = jnp.concatenate([out1, out2], axis=0)
    return jnp.transpose(out_t.reshape(n_fields, dim, batch), (2, 0, 1))


# speedup vs baseline: 3.4012x; 1.0293x over previous
import functools

import jax
import jax.numpy as jnp
from jax import lax
from jax.experimental import pallas as pl
from jax.experimental.pallas import tpu as pltpu
from jax.experimental.pallas import tpu_sc as plsc

CHUNK = 4096


def kernel(lS_i, lS_o, W):
    del lS_o
    n_fields, batch = lS_i.shape
    _, vocab, dim = W.shape

    info = plsc.get_sparse_core_info()
    nw = info.num_cores * info.num_subcores
    d_oct = dim // 8
    nblocks = n_fields * d_oct
    tasks_per_w = (nblocks + nw - 1) // nw

    aligned_end = (vocab // 128) * 128
    sizes = [CHUNK] * (aligned_end // CHUNK)
    if aligned_end % CHUNK:
        sizes.append(aligned_end % CHUNK)
    tail_lo = vocab - 128
    n_sweeps = len(sizes) + 1

    bag_iota = lax.broadcasted_iota(jnp.int32, (n_fields, batch), 1)
    comb = lS_i * batch + bag_iota
    bounds = []
    c0 = 0
    for sz in sizes:
        bounds.append(c0)
        c0 += sz
    bounds.append(aligned_end)

    wt = jnp.transpose(W, (0, 2, 1))
    tail_t = jnp.transpose(W[:, tail_lo:, :], (0, 2, 1))
    bag_shift = batch.bit_length() - 1

    mesh = plsc.VectorSubcoreMesh(core_axis_name="c", subcore_axis_name="s")

    def make_call(f0, nf):
        nblk = nf * d_oct
        tpw = (nblk + nw - 1) // nw

        @functools.partial(
            pl.kernel,
            out_type=jax.ShapeDtypeStruct((nblk * 8, batch), jnp.float32),
            mesh=mesh,
            compiler_params=pltpu.CompilerParams(needs_layout_passes=False),
            scratch_types=[
                pltpu.VMEM((2, 8, CHUNK), jnp.float32),
                pltpu.VMEM((batch + 16,), jnp.int32),
                pltpu.VMEM((8, batch), jnp.float32),
                pltpu.SemaphoreType.DMA,
            ],
        )
        def sc_kernel(scomb_hbm, table_hbm, tail_hbm, out_hbm,
                      buf2, scomb_v, acc, sem):
            wid = lax.axis_index("s") * info.num_cores + lax.axis_index("c")

            def task(t, _):
                block = wid + t * nw

                @pl.when(block < nblk)
                def _():
                    f = f0 + lax.shift_right_logical(block, 2)
                    db = lax.bitwise_and(block, 3)
                    f_local = lax.shift_right_logical(block, 2)
                    pltpu.sync_copy(
                        scomb_hbm.at[pl.ds(f_local * batch, batch)],
                        scomb_v.at[pl.ds(0, batch)],
                    )

                    def stage(c, slot):
                        if c < len(sizes):
                            return pltpu.async_copy(
                                table_hbm.at[
                                    f, pl.ds(db * 8, 8),
                                    pl.ds(bounds[c], sizes[c]),
                                ],
                                buf2.at[slot, :, pl.ds(0, sizes[c])],
                                sem,
                            )
                        return pltpu.async_copy(
                            tail_hbm.at[f, pl.ds(db * 8, 8), :],
                            buf2.at[slot, :, pl.ds(0, 128)],
                            sem,
                        )

                    def sweep(c, slot, p0):
                        base = bounds[c] if c < len(sizes) else tail_lo
                        sz = sizes[c] if c < len(sizes) else 128
                        lo_cut = bounds[c] * batch
                        hi_cut = (
                            (bounds[c] + sizes[c]) * batch
                            if c < len(sizes) else jnp.int32(2**31 - 1)
                        )
                        src = buf2.at[slot, :, pl.ds(0, sz)]

                        def cond(carry):
                            return carry[1]

                        def body(carry):
                            p, _ = carry
                            cv = scomb_v[pl.ds(p, 16)]
                            below = cv < hi_cut
                            msk = jnp.logical_and(cv >= lo_cut, below)
                            iv = lax.shift_right_logical(cv, bag_shift)
                            bagv = lax.bitwise_and(cv, batch - 1)
                            loc = jnp.clip(iv - base, 0, sz - 1)
                            for d in range(8):
                                dv = jnp.full((16,), d, jnp.int32)
                                v = plsc.load_gather(src, [dv, loc])
                                plsc.store_scatter(
                                    acc, [dv, bagv], v, mask=msk
                                )
                            all_in = plsc.all_reduce_population_count(
                                below
                            )[0]
                            p_new = jnp.where(all_in == 16, p + 16, p)
                            cont = jnp.logical_and(
                                all_in == 16, p_new < batch
                            )
                            return (p_new, cont)

                        p_end, _ = lax.while_loop(
                            cond, body, (p0, p0 < batch)
                        )
                        return p_end

                    handles = [stage(0, 0)]
                    pos = jnp.int32(0)
                    for c in range(n_sweeps):
                        handles[c].wait()
                        if c + 1 < n_sweeps:
                            handles.append(stage(c + 1, (c + 1) & 1))
                        pos = sweep(c, c & 1, pos)

                    pltpu.sync_copy(acc, out_hbm.at[pl.ds(block * 8, 8), :])

                return 0

            lax.fori_loop(0, tpw, task, 0)

        return sc_kernel

    splits = [(0, 8), (8, 8), (16, n_fields - 16)]
    parts = []
    for f0, nf in splits:
        s = lax.sort(comb[f0:f0 + nf], dimension=1).reshape(nf * batch)
        o = make_call(f0, nf)(s, wt, tail_t)
        parts.append(
            jnp.transpose(o.reshape(nf, dim, batch), (2, 0, 1))
        )
    return jnp.concatenate(parts, axis=1)
